# baseline (speedup 1.0000x reference)
import jax
import jax.numpy as jnp
from jax import lax
from jax.experimental import pallas as pl
from jax.experimental.pallas import tpu as pltpu
from jax.experimental.pallas import tpu_sc as plsc

_NC = 2
_NS = 16
_CHUNK = 128


def _sc_aggregate(x, src2d, dst2d, n_pad):
    n, d = x.shape
    n_chunks = dst2d.shape[0]
    cpw = n_chunks // (_NC * _NS)
    hc = cpw // 2
    rpt = n_pad // _NS
    nz = rpt // _CHUNK

    mesh = plsc.VectorSubcoreMesh(core_axis_name="c", subcore_axis_name="s")

    def body(x_hbm, src2d_hbm, dst2d_hbm, out_hbm, src_v, dst_v,
             rows0, rows1, sem0, sem1, agg_sh):
        c = lax.axis_index("c")
        s = lax.axis_index("s")
        w = c * _NS + s
        start = w * cpw

        def gather(j, buf, sem):
            return pltpu.async_copy(x_hbm.at[src_v.at[j]], buf, sem)

        def pair(i, carry):
            j = 2 * i
            pltpu.make_async_copy(x_hbm.at[src_v.at[j]], rows0, sem0).wait()
            gather(j + 1, rows1, sem1)
            pltpu.sync_copy(rows0, agg_sh.at[dst_v.at[j]], add=True)
            pltpu.make_async_copy(x_hbm.at[src_v.at[j + 1]], rows1, sem1).wait()

            @pl.when(j + 2 < hc)
            def _():
                gather(j + 2, rows0, sem0)
            pltpu.sync_copy(rows1, agg_sh.at[dst_v.at[j + 1]], add=True)
            return carry

        for half in range(2):
            pltpu.sync_copy(src2d_hbm.at[pl.ds(start + half * hc, hc)], src_v)
            pltpu.sync_copy(dst2d_hbm.at[pl.ds(start + half * hc, hc)], dst_v)
            if half == 0:
                def zstore(i, carry):
                    for j in range(d // 16):
                        rows0[i, pl.ds(j * 16, 16)] = jnp.zeros((16,), jnp.float32)
                    return carry
                lax.fori_loop(0, _CHUNK, zstore, 0)
                for k in range(nz):
                    pltpu.sync_copy(rows0, agg_sh.at[pl.ds(s * rpt + k * _CHUNK, _CHUNK)])
                plsc.subcore_barrier()
            gather(0, rows0, sem0)
            lax.fori_loop(0, hc // 2, pair, 0)
        plsc.subcore_barrier()

        pltpu.sync_copy(agg_sh.at[pl.ds(s * rpt, rpt)], out_hbm.at[c, pl.ds(s * rpt, rpt)])

    run = pl.kernel(
        body,
        out_type=jax.ShapeDtypeStruct((_NC, n_pad, d), jnp.float32),
        mesh=mesh,
        scratch_types=[
            pltpu.VMEM((hc, _CHUNK), jnp.int32),
            pltpu.VMEM((hc, _CHUNK), jnp.int32),
            pltpu.VMEM((_CHUNK, d), jnp.float32),
            pltpu.VMEM((_CHUNK, d), jnp.float32),
            pltpu.SemaphoreType.DMA,
            pltpu.SemaphoreType.DMA,
            pltpu.VMEM_SHARED((n_pad, d), jnp.float32),
        ],
    )
    return run(x, src2d, dst2d)


def _bn_relu_mlp(h, wa, ba, g, be, wb, bb):
    t = jnp.dot(h, wa, preferred_element_type=jnp.float32) + ba
    m = jnp.mean(t, axis=0, keepdims=True)
    v = jnp.mean((t - m) ** 2, axis=0, keepdims=True)
    t = g * (t - m) * lax.rsqrt(v + 1e-5) + be
    t = jnp.maximum(t, 0.0)
    t = jnp.dot(t, wb, preferred_element_type=jnp.float32) + bb
    return jnp.maximum(t, 0.0)


def _tc_conv1(x, parts, wa, ba, g, be, wb, bb):
    n, d = x.shape

    def body(x_ref, p_ref, wa_ref, ba_ref, g_ref, be_ref, wb_ref, bb_ref, o_ref):
        h = x_ref[...] + p_ref[0, :n] + p_ref[1, :n]
        o_ref[...] = _bn_relu_mlp(h, wa_ref[...], ba_ref[...], g_ref[...],
                                  be_ref[...], wb_ref[...], bb_ref[...])

    return pl.pallas_call(
        body, out_shape=jax.ShapeDtypeStruct((n, d), jnp.float32)
    )(x, parts, wa, ba.reshape(1, -1), g.reshape(1, -1), be.reshape(1, -1),
      wb, bb.reshape(1, -1))


def _tc_conv2_final(h1, parts, wa, ba, g, be, wb, bb, wf, bf):
    n, d = h1.shape
    ncls = wf.shape[1]

    def body(h_ref, p_ref, wa_ref, ba_ref, g_ref, be_ref, wb_ref, bb_ref,
             wf_ref, bf_ref, o_ref):
        h = h_ref[...] + p_ref[0, :n] + p_ref[1, :n]
        h2 = _bn_relu_mlp(h, wa_ref[...], ba_ref[...], g_ref[...],
                          be_ref[...], wb_ref[...], bb_ref[...])
        o = jnp.dot(h2, wf_ref[...], preferred_element_type=jnp.float32) + bf_ref[...]
        mx = jnp.max(o, axis=1, keepdims=True)
        o = o - mx
        o_ref[...] = o - jnp.log(jnp.sum(jnp.exp(o), axis=1, keepdims=True))

    return pl.pallas_call(
        body, out_shape=jax.ShapeDtypeStruct((n, ncls), jnp.float32)
    )(h1, parts, wa, ba.reshape(1, -1), g.reshape(1, -1), be.reshape(1, -1),
      wb, bb.reshape(1, -1), wf, bf.reshape(1, -1))


def kernel(x, edge_index, w1a, b1a, g1, be1, w1b, b1b,
           w2a, b2a, g2, be2, w2b, b2b, wf, bf):
    n, d = x.shape
    e = edge_index.shape[1]
    workers = _NC * _NS
    cpw = -(-e // (_CHUNK * workers))
    cpw = (cpw + 7) // 8 * 8
    e_pad = cpw * _CHUNK * workers

    n_pad = -(-n // 1024) * 1024

    src = edge_index[0].astype(jnp.int32)
    dst = edge_index[1].astype(jnp.int32)
    if e_pad != e:
        pad = e_pad - e
        src = jnp.concatenate([src, jnp.zeros((pad,), jnp.int32)])
        dummy = n + jnp.arange(pad, dtype=jnp.int32) % jnp.int32(n_pad - n)
        dst = jnp.concatenate([dst, dummy])
    src2d = src.reshape(-1, _CHUNK)
    dst2d = dst.reshape(-1, _CHUNK)

    parts1 = _sc_aggregate(x, src2d, dst2d, n_pad)
    h1 = _tc_conv1(x, parts1, w1a, b1a, g1, be1, w1b, b1b)
    parts2 = _sc_aggregate(h1, src2d, dst2d, n_pad)
    return _tc_conv2_final(h1, parts2, w2a, b2a, g2, be2, w2b, b2b, wf, bf)


# speedup vs baseline: 3.5302x; 3.5302x over previous
import jax
import jax.numpy as jnp
from jax import lax
from jax.experimental import pallas as pl
from jax.experimental.pallas import tpu as pltpu
from jax.experimental.pallas import tpu_sc as plsc

_NC = 2
_NS = 16
_CHUNK = 128


def _sc_aggregate(x, src2d, dst2d, n_pad):
    n, d = x.shape
    n_chunks = dst2d.shape[0]
    cpw = n_chunks // (_NC * _NS)
    hc = cpw // 2
    rpt = n_pad // _NS
    nz = rpt // _CHUNK

    mesh = plsc.VectorSubcoreMesh(core_axis_name="c", subcore_axis_name="s")

    def body(x_hbm, src2d_hbm, dst2d_hbm, out_hbm, src_v, dst_v,
             rows0, rows1, sem0, sem1, agg_sh):
        c = lax.axis_index("c")
        s = lax.axis_index("s")
        w = c * _NS + s
        start = w * cpw

        def gather(j, buf, sem):
            return pltpu.async_copy(x_hbm.at[src_v.at[j]], buf, sem)

        def pair(i, carry):
            j = 2 * i
            pltpu.make_async_copy(x_hbm.at[src_v.at[j]], rows0, sem0).wait()
            gather(j + 1, rows1, sem1)
            pltpu.sync_copy(rows0, agg_sh.at[dst_v.at[j]], add=True)
            pltpu.make_async_copy(x_hbm.at[src_v.at[j + 1]], rows1, sem1).wait()

            @pl.when(j + 2 < hc)
            def _():
                gather(j + 2, rows0, sem0)
            pltpu.sync_copy(rows1, agg_sh.at[dst_v.at[j + 1]], add=True)
            return carry

        for half in range(2):
            pltpu.sync_copy(src2d_hbm.at[pl.ds(start + half * hc, hc)], src_v)
            pltpu.sync_copy(dst2d_hbm.at[pl.ds(start + half * hc, hc)], dst_v)
            if half == 0:
                def zstore(i, carry):
                    for j in range(d // 16):
                        rows0[i, pl.ds(j * 16, 16)] = jnp.zeros((16,), jnp.float32)
                    return carry
                lax.fori_loop(0, _CHUNK, zstore, 0)
                for k in range(nz):
                    pltpu.sync_copy(rows0, agg_sh.at[pl.ds(s * rpt + k * _CHUNK, _CHUNK)])
                plsc.subcore_barrier()
            gather(0, rows0, sem0)
            lax.fori_loop(0, hc // 2, pair, 0)
        plsc.subcore_barrier()

        pltpu.sync_copy(agg_sh.at[pl.ds(s * rpt, rpt)], out_hbm.at[c, pl.ds(s * rpt, rpt)])

    run = pl.kernel(
        body,
        out_type=jax.ShapeDtypeStruct((_NC, n_pad, d), jnp.float32),
        mesh=mesh,
        scratch_types=[
            pltpu.VMEM((hc, _CHUNK), jnp.int32),
            pltpu.VMEM((hc, _CHUNK), jnp.int32),
            pltpu.VMEM((_CHUNK, d), jnp.float32),
            pltpu.VMEM((_CHUNK, d), jnp.float32),
            pltpu.SemaphoreType.DMA,
            pltpu.SemaphoreType.DMA,
            pltpu.VMEM_SHARED((n_pad, d), jnp.float32),
        ],
    )
    return run(x, src2d, dst2d)


def _bn_relu_mlp(h, wa, ba, g, be, wb, bb):
    t = jnp.dot(h, wa, preferred_element_type=jnp.float32) + ba
    m = jnp.mean(t, axis=0, keepdims=True)
    v = jnp.mean((t - m) ** 2, axis=0, keepdims=True)
    t = g * (t - m) * lax.rsqrt(v + 1e-5) + be
    t = jnp.maximum(t, 0.0)
    t = jnp.dot(t, wb, preferred_element_type=jnp.float32) + bb
    return jnp.maximum(t, 0.0)


def _tc_conv1(x, parts, wa, ba, g, be, wb, bb):
    n, d = x.shape

    def body(x_ref, p_ref, wa_ref, ba_ref, g_ref, be_ref, wb_ref, bb_ref, o_ref):
        h = x_ref[...] + p_ref[0, :n] + p_ref[1, :n]
        o_ref[...] = _bn_relu_mlp(h, wa_ref[...], ba_ref[...], g_ref[...],
                                  be_ref[...], wb_ref[...], bb_ref[...])

    return pl.pallas_call(
        body, out_shape=jax.ShapeDtypeStruct((n, d), jnp.float32)
    )(x, parts, wa, ba.reshape(1, -1), g.reshape(1, -1), be.reshape(1, -1),
      wb, bb.reshape(1, -1))


def _tc_conv2_final(h1, parts, wa, ba, g, be, wb, bb, wf, bf):
    n, d = h1.shape
    ncls = wf.shape[1]

    def body(h_ref, p_ref, wa_ref, ba_ref, g_ref, be_ref, wb_ref, bb_ref,
             wf_ref, bf_ref, o_ref):
        h = h_ref[...] + p_ref[0, :n] + p_ref[1, :n]
        h2 = _bn_relu_mlp(h, wa_ref[...], ba_ref[...], g_ref[...],
                          be_ref[...], wb_ref[...], bb_ref[...])
        o = jnp.dot(h2, wf_ref[...], preferred_element_type=jnp.float32) + bf_ref[...]
        mx = jnp.max(o, axis=1, keepdims=True)
        o = o - mx
        o_ref[...] = o - jnp.log(jnp.sum(jnp.exp(o), axis=1, keepdims=True))

    return pl.pallas_call(
        body, out_shape=jax.ShapeDtypeStruct((n, ncls), jnp.float32)
    )(h1, parts, wa, ba.reshape(1, -1), g.reshape(1, -1), be.reshape(1, -1),
      wb, bb.reshape(1, -1), wf, bf.reshape(1, -1))


def kernel(x, edge_index, w1a, b1a, g1, be1, w1b, b1b,
           w2a, b2a, g2, be2, w2b, b2b, wf, bf):
    n, d = x.shape
    e = edge_index.shape[1]
    workers = _NC * _NS
    cpw = -(-e // (_CHUNK * workers))
    cpw = (cpw + 7) // 8 * 8
    e_pad = cpw * _CHUNK * workers

    n_pad = -(-n // 1024) * 1024

    src = edge_index[0].astype(jnp.int32)
    dst = edge_index[1].astype(jnp.int32)
    if e_pad != e:
        pad = e_pad - e
        iota = jnp.arange(pad, dtype=jnp.int32)
        src = jnp.concatenate([src, iota % jnp.int32(n)])
        dst = jnp.concatenate([dst, n + iota % jnp.int32(n_pad - n)])
    src2d = src.reshape(-1, _CHUNK)
    dst2d = dst.reshape(-1, _CHUNK)

    parts1 = _sc_aggregate(x, src2d, dst2d, n_pad)
    h1 = _tc_conv1(x, parts1, w1a, b1a, g1, be1, w1b, b1b)
    parts2 = _sc_aggregate(h1, src2d, dst2d, n_pad)
    return _tc_conv2_final(h1, parts2, w2a, b2a, g2, be2, w2b, b2b, wf, bf)


# speedup vs baseline: 4.2230x; 1.1962x over previous
import jax
import jax.numpy as jnp
from jax import lax
from jax.experimental import pallas as pl
from jax.experimental.pallas import tpu as pltpu
from jax.experimental.pallas import tpu_sc as plsc

_NC = 2
_NS = 16
_CHUNK = 64
_NB = 4


def _sc_aggregate(x, src2d, dst2d, n_pad):
    n, d = x.shape
    n_chunks = dst2d.shape[0]
    cpw = n_chunks // (_NC * _NS)
    hc = cpw // 4
    rpt = n_pad // _NS
    nz = rpt // _CHUNK

    mesh = plsc.VectorSubcoreMesh(core_axis_name="c", subcore_axis_name="s")

    def body(x_hbm, src2d_hbm, dst2d_hbm, out_hbm, src_v, dst_v,
             rows0, rows1, rows2, rows3, sem0, sem1, sem2, sem3, agg_sh):
        c = lax.axis_index("c")
        s = lax.axis_index("s")
        w = c * _NS + s
        start = w * cpw
        bufs = (rows0, rows1, rows2, rows3)
        sems = (sem0, sem1, sem2, sem3)

        def gather(j, k):
            pltpu.async_copy(x_hbm.at[src_v.at[j]], bufs[k], sems[k])

        def group(i, carry):
            j = i * _NB
            for k in range(_NB):
                pltpu.make_async_copy(x_hbm.at[src_v.at[j + k]], bufs[k], sems[k]).wait()
                pltpu.sync_copy(bufs[k], agg_sh.at[dst_v.at[j + k]], add=True)

                @pl.when(j + k + _NB < hc)
                def _():
                    gather(j + k + _NB, k)
            return carry

        for quarter in range(4):
            pltpu.sync_copy(src2d_hbm.at[pl.ds(start + quarter * hc, hc)], src_v)
            pltpu.sync_copy(dst2d_hbm.at[pl.ds(start + quarter * hc, hc)], dst_v)
            if quarter == 0:
                def zstore(i, carry):
                    for j in range(d // 16):
                        rows0[i, pl.ds(j * 16, 16)] = jnp.zeros((16,), jnp.float32)
                    return carry
                lax.fori_loop(0, _CHUNK, zstore, 0)
                for k in range(nz):
                    pltpu.sync_copy(rows0, agg_sh.at[pl.ds(s * rpt + k * _CHUNK, _CHUNK)])
                plsc.subcore_barrier()
            for k in range(_NB):
                gather(k, k)
            lax.fori_loop(0, hc // _NB, group, 0)
        plsc.subcore_barrier()

        pltpu.sync_copy(agg_sh.at[pl.ds(s * rpt, rpt)], out_hbm.at[c, pl.ds(s * rpt, rpt)])

    run = pl.kernel(
        body,
        out_type=jax.ShapeDtypeStruct((_NC, n_pad, d), jnp.float32),
        mesh=mesh,
        scratch_types=[
            pltpu.VMEM((hc, _CHUNK), jnp.int32),
            pltpu.VMEM((hc, _CHUNK), jnp.int32),
            pltpu.VMEM((_CHUNK, d), jnp.float32),
            pltpu.VMEM((_CHUNK, d), jnp.float32),
            pltpu.VMEM((_CHUNK, d), jnp.float32),
            pltpu.VMEM((_CHUNK, d), jnp.float32),
            pltpu.SemaphoreType.DMA,
            pltpu.SemaphoreType.DMA,
            pltpu.SemaphoreType.DMA,
            pltpu.SemaphoreType.DMA,
            pltpu.VMEM_SHARED((n_pad, d), jnp.float32),
        ],
    )
    return run(x, src2d, dst2d)


def _bn_relu_mlp(h, wa, ba, g, be, wb, bb):
    t = jnp.dot(h, wa, preferred_element_type=jnp.float32) + ba
    m = jnp.mean(t, axis=0, keepdims=True)
    v = jnp.mean((t - m) ** 2, axis=0, keepdims=True)
    t = g * (t - m) * lax.rsqrt(v + 1e-5) + be
    t = jnp.maximum(t, 0.0)
    t = jnp.dot(t, wb, preferred_element_type=jnp.float32) + bb
    return jnp.maximum(t, 0.0)


def _tc_conv1(x, parts, wa, ba, g, be, wb, bb):
    n, d = x.shape

    def body(x_ref, p_ref, wa_ref, ba_ref, g_ref, be_ref, wb_ref, bb_ref, o_ref):
        h = x_ref[...] + p_ref[0, :n] + p_ref[1, :n]
        o_ref[...] = _bn_relu_mlp(h, wa_ref[...], ba_ref[...], g_ref[...],
                                  be_ref[...], wb_ref[...], bb_ref[...])

    return pl.pallas_call(
        body, out_shape=jax.ShapeDtypeStruct((n, d), jnp.float32)
    )(x, parts, wa, ba.reshape(1, -1), g.reshape(1, -1), be.reshape(1, -1),
      wb, bb.reshape(1, -1))


def _tc_conv2_final(h1, parts, wa, ba, g, be, wb, bb, wf, bf):
    n, d = h1.shape
    ncls = wf.shape[1]

    def body(h_ref, p_ref, wa_ref, ba_ref, g_ref, be_ref, wb_ref, bb_ref,
             wf_ref, bf_ref, o_ref):
        h = h_ref[...] + p_ref[0, :n] + p_ref[1, :n]
        h2 = _bn_relu_mlp(h, wa_ref[...], ba_ref[...], g_ref[...],
                          be_ref[...], wb_ref[...], bb_ref[...])
        o = jnp.dot(h2, wf_ref[...], preferred_element_type=jnp.float32) + bf_ref[...]
        mx = jnp.max(o, axis=1, keepdims=True)
        o = o - mx
        o_ref[...] = o - jnp.log(jnp.sum(jnp.exp(o), axis=1, keepdims=True))

    return pl.pallas_call(
        body, out_shape=jax.ShapeDtypeStruct((n, ncls), jnp.float32)
    )(h1, parts, wa, ba.reshape(1, -1), g.reshape(1, -1), be.reshape(1, -1),
      wb, bb.reshape(1, -1), wf, bf.reshape(1, -1))


def kernel(x, edge_index, w1a, b1a, g1, be1, w1b, b1b,
           w2a, b2a, g2, be2, w2b, b2b, wf, bf):
    n, d = x.shape
    e = edge_index.shape[1]
    workers = _NC * _NS
    cpw = -(-e // (_CHUNK * workers))
    cpw = (cpw + 31) // 32 * 32
    e_pad = cpw * _CHUNK * workers

    n_pad = -(-n // 1024) * 1024

    src = edge_index[0].astype(jnp.int32)
    dst = edge_index[1].astype(jnp.int32)
    if e_pad != e:
        pad = e_pad - e
        iota = jnp.arange(pad, dtype=jnp.int32)
        src = jnp.concatenate([src, iota % jnp.int32(n)])
        dst = jnp.concatenate([dst, n + iota % jnp.int32(n_pad - n)])
    src2d = src.reshape(-1, _CHUNK)
    dst2d = dst.reshape(-1, _CHUNK)

    parts1 = _sc_aggregate(x, src2d, dst2d, n_pad)
    h1 = _tc_conv1(x, parts1, w1a, b1a, g1, be1, w1b, b1b)
    parts2 = _sc_aggregate(h1, src2d, dst2d, n_pad)
    return _tc_conv2_final(h1, parts2, w2a, b2a, g2, be2, w2b, b2b, wf, bf)
